# Initial kernel scaffold; baseline (speedup 1.0000x reference)
import jax
import jax.numpy as jnp
from jax.experimental import pallas as pl


def kernel(x, edge_index, W_self0, W_neigh0, b0, W_self1, W_neigh1, b1):
    raise NotImplementedError("write your pallas kernel here")


# speedup vs baseline: 2.8139x; 2.8139x over previous
import functools

import jax
import jax.numpy as jnp
from jax import lax
from jax.experimental import pallas as pl
from jax.experimental.pallas import tpu as pltpu
from jax.experimental.pallas import tpu_sc as plsc

D = 128
CHUNK = 128
NW = 32
NSUB = 16
L = 16


def _round_up(a, b):
    return (a + b - 1) // b * b


@functools.lru_cache(maxsize=None)
def _make_sc_agg(n_pad, cpw, with_deg):
    rpt = n_pad // NSUB
    mesh = plsc.VectorSubcoreMesh(core_axis_name="c", subcore_axis_name="s")

    out_type = [jax.ShapeDtypeStruct((2, n_pad, D), jnp.float32)]
    scratch = [
        pltpu.VMEM((cpw, CHUNK), jnp.int32),
        pltpu.VMEM((cpw, CHUNK), jnp.int32),
        pltpu.VMEM((CHUNK, D), jnp.float32),
        pltpu.VMEM_SHARED((n_pad, D), jnp.float32),
    ]
    nrow = n_pad // CHUNK
    if with_deg:
        out_type.append(jax.ShapeDtypeStruct((2, nrow, CHUNK), jnp.float32))
        scratch += [
            pltpu.VMEM((nrow, CHUNK), jnp.float32),
            pltpu.VMEM_SHARED((nrow, CHUNK), jnp.float32),
            pltpu.VMEM((nrow,), jnp.int32),
        ]

    @functools.partial(pl.kernel, out_type=out_type, mesh=mesh,
                       scratch_types=scratch,
                       compiler_params=pltpu.CompilerParams(
                           needs_layout_passes=False))
    def sc_agg(feats, srcc, dstc, zeros, out, *rest):
        if with_deg:
            out_deg, src_v, dst_v, rows_v, acc_sh, deg_v, deg_sh, iota_v = rest
        else:
            src_v, dst_v, rows_v, acc_sh = rest
        cid = lax.axis_index("c")
        sid = lax.axis_index("s")
        wid = sid * 2 + cid

        pltpu.sync_copy(zeros.at[pl.ds(sid * rpt, rpt)],
                        acc_sh.at[pl.ds(sid * rpt, rpt)])
        pltpu.sync_copy(srcc.at[pl.ds(wid * cpw, cpw)], src_v)
        pltpu.sync_copy(dstc.at[pl.ds(wid * cpw, cpw)], dst_v)

        if with_deg:
            pltpu.sync_copy(zeros.at[pl.ds(0, nrow)], deg_v)

            def istep(i, c):
                iota_v[pl.ds(i * L, L)] = lax.iota(jnp.int32, L) + i * L
                return c

            lax.fori_loop(0, nrow // L, istep, 0)

            @pl.when(sid == 0)
            def _():
                pltpu.sync_copy(zeros.at[pl.ds(0, nrow)], deg_sh)

        plsc.subcore_barrier()

        def step(j, c):
            pltpu.sync_copy(feats.at[src_v.at[j]], rows_v)
            pltpu.sync_copy(rows_v, acc_sh.at[dst_v.at[j]], add=True)
            return c

        lax.fori_loop(0, cpw, step, 0)

        if with_deg:
            ones16 = jnp.ones((L,), jnp.float32)

            def dstep(j, c):
                for k in range(CHUNK // L):
                    idx = dst_v[j, pl.ds(k * L, L)]
                    plsc.addupdate_scatter(
                        deg_v,
                        [lax.shift_right_logical(idx, 7), idx & 127],
                        ones16)
                return c

            lax.fori_loop(0, cpw, dstep, 0)
            pltpu.sync_copy(deg_v, deg_sh.at[iota_v], add=True)

        plsc.subcore_barrier()

        pltpu.sync_copy(acc_sh.at[pl.ds(sid * rpt, rpt)],
                        out.at[cid, pl.ds(sid * rpt, rpt)])
        if with_deg:
            @pl.when(sid == 0)
            def _():
                pltpu.sync_copy(deg_sh, out_deg.at[cid])

    return sc_agg


@functools.lru_cache(maxsize=None)
def _make_tc_layer(n, n_pad, rows):
    grid = (n // rows,)

    def body(h_ref, acc_ref, deg_ref, ws_ref, wn_ref, b_ref, out_ref):
        agg = acc_ref[0] + acc_ref[1]
        deg = deg_ref[0] + deg_ref[1]
        mean = agg / jnp.maximum(deg, 1.0)
        o = h_ref[...] @ ws_ref[...] + mean @ wn_ref[...] + b_ref[...]
        out_ref[...] = jnp.maximum(o, 0.0)

    return pl.pallas_call(
        body,
        grid=grid,
        in_specs=[
            pl.BlockSpec((rows, D), lambda i: (i, 0)),
            pl.BlockSpec((2, rows, D), lambda i: (0, i, 0)),
            pl.BlockSpec((2, rows, 1), lambda i: (0, i, 0)),
            pl.BlockSpec((D, D), lambda i: (0, 0)),
            pl.BlockSpec((D, D), lambda i: (0, 0)),
            pl.BlockSpec((1, D), lambda i: (0, 0)),
        ],
        out_specs=pl.BlockSpec((rows, D), lambda i: (i, 0)),
        out_shape=jax.ShapeDtypeStruct((n, D), jnp.float32),
    )


def kernel(x, edge_index, W_self0, W_neigh0, b0, W_self1, W_neigh1, b1):
    n = x.shape[0]
    e = edge_index.shape[1]
    n_pad = _round_up(n + 1, NSUB * L)
    e_pad = _round_up(e, NW * CHUNK * 8)
    cpw = e_pad // (NW * CHUNK)

    src = edge_index[0]
    dst = edge_index[1]
    pad = e_pad - e
    srcc = jnp.concatenate([src, jnp.zeros((pad,), jnp.int32)]).reshape(-1, CHUNK)
    dstc = jnp.concatenate([dst, jnp.full((pad,), n, jnp.int32)]).reshape(-1, CHUNK)
    zeros = jnp.zeros((n_pad, D), jnp.float32)

    sc_agg1 = _make_sc_agg(n_pad, cpw, True)
    sc_agg2 = _make_sc_agg(n_pad, cpw, False)
    rows = 2000 if n % 2000 == 0 else n
    tc_layer = _make_tc_layer(n, n_pad, rows)

    agg1, deg = sc_agg1(x, srcc, dstc, zeros)
    deg = deg.reshape(2, n_pad, 1)
    h1 = tc_layer(x, agg1, deg, W_self0, W_neigh0, b0.reshape(1, D))
    (agg2,) = sc_agg2(h1, srcc, dstc, zeros)
    h2 = tc_layer(h1, agg2, deg, W_self1, W_neigh1, b1.reshape(1, D))
    return h2


# speedup vs baseline: 3.0252x; 1.0751x over previous
import functools

import jax
import jax.numpy as jnp
from jax import lax
from jax.experimental import pallas as pl
from jax.experimental.pallas import tpu as pltpu
from jax.experimental.pallas import tpu_sc as plsc

D = 128
CHUNK = 128
NW = 32
NSUB = 16
L = 16
NBUF = 2
NPHASE = 2


def _round_up(a, b):
    return (a + b - 1) // b * b


@functools.lru_cache(maxsize=None)
def _make_sc_agg(n_pad, cpw, with_deg):
    rpt = n_pad // NSUB
    mesh = plsc.VectorSubcoreMesh(core_axis_name="c", subcore_axis_name="s")

    out_type = [jax.ShapeDtypeStruct((2, n_pad, D), jnp.float32)]
    hc = cpw // NPHASE
    scratch = [
        pltpu.VMEM((hc, CHUNK), jnp.int32),
        pltpu.VMEM((hc, CHUNK), jnp.int32),
        pltpu.VMEM((NBUF, CHUNK, D), jnp.float32),
        pltpu.VMEM_SHARED((n_pad, D), jnp.float32),
    ] + [pltpu.SemaphoreType.DMA] * (2 * NBUF)
    nrow = n_pad // CHUNK
    if with_deg:
        out_type.append(jax.ShapeDtypeStruct((2, nrow, CHUNK), jnp.float32))
        scratch += [
            pltpu.VMEM_SHARED((nrow, CHUNK), jnp.float32),
            pltpu.VMEM((nrow,), jnp.int32),
        ]

    @functools.partial(pl.kernel, out_type=out_type, mesh=mesh,
                       scratch_types=scratch,
                       compiler_params=pltpu.CompilerParams(
                           needs_layout_passes=False))
    def sc_agg(feats, srcc, dstc, zeros, out, *rest):
        if with_deg:
            out_deg = rest[0]
            rest = rest[1:]
        src_v, dst_v, rows_v, acc_sh = rest[:4]
        gsem = rest[4:4 + NBUF]
        ssem = rest[4 + NBUF:4 + 2 * NBUF]
        if with_deg:
            deg_sh, iota_v = rest[4 + 2 * NBUF:]
        cid = lax.axis_index("c")
        sid = lax.axis_index("s")
        wid = sid * 2 + cid

        pltpu.sync_copy(zeros.at[pl.ds(sid * rpt, rpt)],
                        acc_sh.at[pl.ds(sid * rpt, rpt)])

        if with_deg:
            hist = rows_v.at[0, pl.ds(0, nrow)]
            pltpu.sync_copy(zeros.at[pl.ds(0, nrow)], hist)

            def istep(i, c):
                iota_v[pl.ds(i * L, L)] = lax.iota(jnp.int32, L) + i * L
                return c

            lax.fori_loop(0, nrow // L, istep, 0)

            @pl.when(sid == 0)
            def _():
                pltpu.sync_copy(zeros.at[pl.ds(0, nrow)], deg_sh)

        plsc.subcore_barrier()

        if with_deg:
            ones16 = jnp.ones((L,), jnp.float32)
            zeros16i = jnp.zeros((L,), jnp.int32)
            for p in range(NPHASE):
                pltpu.sync_copy(dstc.at[pl.ds(wid * cpw + p * hc, hc)], dst_v)

                def dstep(j, c):
                    for k in range(CHUNK // L):
                        idx = dst_v[j, pl.ds(k * L, L)]
                        plsc.addupdate_scatter(
                            rows_v,
                            [zeros16i, lax.shift_right_logical(idx, 7),
                             idx & 127],
                            ones16)
                    return c

                lax.fori_loop(0, hc, dstep, 0)
            pltpu.sync_copy(hist, deg_sh.at[iota_v], add=True)

        for p in range(NPHASE):
            pltpu.sync_copy(srcc.at[pl.ds(wid * cpw + p * hc, hc)], src_v)
            pltpu.sync_copy(dstc.at[pl.ds(wid * cpw + p * hc, hc)], dst_v)
            for b in range(NBUF):
                pltpu.async_copy(feats.at[src_v.at[b]], rows_v.at[b], gsem[b])

            def step(i, c):
                t = i * NBUF
                for b in range(NBUF):
                    j = t + b
                    pltpu.make_async_copy(
                        feats.at[src_v.at[j]], rows_v.at[b], gsem[b]).wait()
                    pltpu.async_copy(rows_v.at[b], acc_sh.at[dst_v.at[j]],
                                     ssem[b], add=True)
                for b in range(NBUF):
                    j = t + b
                    pltpu.make_async_copy(
                        rows_v.at[b], acc_sh.at[dst_v.at[j]], ssem[b]).wait()
                    jn = j + NBUF

                    @pl.when(jn < hc)
                    def _():
                        pltpu.async_copy(feats.at[src_v.at[jn]],
                                         rows_v.at[b], gsem[b])
                return c

            lax.fori_loop(0, hc // NBUF, step, 0)

        plsc.subcore_barrier()

        pltpu.sync_copy(acc_sh.at[pl.ds(sid * rpt, rpt)],
                        out.at[cid, pl.ds(sid * rpt, rpt)])
        if with_deg:
            @pl.when(sid == 0)
            def _():
                pltpu.sync_copy(deg_sh, out_deg.at[cid])

    return sc_agg


@functools.lru_cache(maxsize=None)
def _make_tc_layer(n, n_pad, rows):
    grid = (n // rows,)

    def body(h_ref, acc_ref, deg_ref, ws_ref, wn_ref, b_ref, out_ref):
        agg = acc_ref[0] + acc_ref[1]
        deg = deg_ref[0] + deg_ref[1]
        mean = agg / jnp.maximum(deg, 1.0)
        o = h_ref[...] @ ws_ref[...] + mean @ wn_ref[...] + b_ref[...]
        out_ref[...] = jnp.maximum(o, 0.0)

    return pl.pallas_call(
        body,
        grid=grid,
        in_specs=[
            pl.BlockSpec((rows, D), lambda i: (i, 0)),
            pl.BlockSpec((2, rows, D), lambda i: (0, i, 0)),
            pl.BlockSpec((2, rows, 1), lambda i: (0, i, 0)),
            pl.BlockSpec((D, D), lambda i: (0, 0)),
            pl.BlockSpec((D, D), lambda i: (0, 0)),
            pl.BlockSpec((1, D), lambda i: (0, 0)),
        ],
        out_specs=pl.BlockSpec((rows, D), lambda i: (i, 0)),
        out_shape=jax.ShapeDtypeStruct((n, D), jnp.float32),
    )


def kernel(x, edge_index, W_self0, W_neigh0, b0, W_self1, W_neigh1, b1):
    n = x.shape[0]
    e = edge_index.shape[1]
    n_pad = _round_up(n + 1, NSUB * L)
    e_pad = _round_up(e, NW * CHUNK * NPHASE * 8)
    cpw = e_pad // (NW * CHUNK)

    src = edge_index[0]
    dst = edge_index[1]
    pad = e_pad - e
    srcc = jnp.concatenate([src, jnp.zeros((pad,), jnp.int32)]).reshape(-1, CHUNK)
    dstc = jnp.concatenate([dst, jnp.full((pad,), n, jnp.int32)]).reshape(-1, CHUNK)
    zeros = jnp.zeros((n_pad, D), jnp.float32)

    sc_agg1 = _make_sc_agg(n_pad, cpw, True)
    sc_agg2 = _make_sc_agg(n_pad, cpw, False)
    rows = 2000 if n % 2000 == 0 else n
    tc_layer = _make_tc_layer(n, n_pad, rows)

    agg1, deg = sc_agg1(x, srcc, dstc, zeros)
    deg = deg.reshape(2, n_pad, 1)
    h1 = tc_layer(x, agg1, deg, W_self0, W_neigh0, b0.reshape(1, D))
    (agg2,) = sc_agg2(h1, srcc, dstc, zeros)
    h2 = tc_layer(h1, agg2, deg, W_self1, W_neigh1, b1.reshape(1, D))
    return h2


# speedup vs baseline: 3.4577x; 1.1430x over previous
import functools

import jax
import jax.numpy as jnp
from jax import lax
from jax.experimental import pallas as pl
from jax.experimental.pallas import tpu as pltpu
from jax.experimental.pallas import tpu_sc as plsc

D = 128
CHUNK = 128
NW = 32
NSUB = 16
L = 16
NBUF = 2
NPHASE = 2


def _round_up(a, b):
    return (a + b - 1) // b * b


@functools.lru_cache(maxsize=None)
def _make_sc_agg(n_pad, cpw, with_deg):
    rpt = n_pad // NSUB
    mesh = plsc.VectorSubcoreMesh(core_axis_name="c", subcore_axis_name="s")

    out_type = [jax.ShapeDtypeStruct((2, n_pad, D), jnp.float32)]
    hc = cpw // NPHASE
    scratch = [
        pltpu.VMEM((hc, CHUNK), jnp.int32),
        pltpu.VMEM((hc, CHUNK), jnp.int32),
        pltpu.VMEM((NBUF, CHUNK, D), jnp.float32),
        pltpu.VMEM_SHARED((n_pad, D), jnp.float32),
    ] + [pltpu.SemaphoreType.DMA] * (2 * NBUF)
    nrow = n_pad // CHUNK
    if with_deg:
        out_type.append(jax.ShapeDtypeStruct((2, nrow, CHUNK), jnp.float32))
        scratch += [
            pltpu.VMEM_SHARED((nrow, CHUNK), jnp.float32),
            pltpu.VMEM((nrow,), jnp.int32),
        ]

    @functools.partial(pl.kernel, out_type=out_type, mesh=mesh,
                       scratch_types=scratch,
                       compiler_params=pltpu.CompilerParams(
                           needs_layout_passes=False))
    def sc_agg(feats, srcc, dstc, out, *rest):
        if with_deg:
            out_deg = rest[0]
            rest = rest[1:]
        src_v, dst_v, rows_v, acc_sh = rest[:4]
        gsem = rest[4:4 + NBUF]
        ssem = rest[4 + NBUF:4 + 2 * NBUF]
        if with_deg:
            deg_sh, iota_v = rest[4 + 2 * NBUF:]
        cid = lax.axis_index("c")
        sid = lax.axis_index("s")
        wid = sid * 2 + cid

        z16 = jnp.zeros((L,), jnp.float32)

        def zstep(r, c):
            for k in range(D // L):
                rows_v[0, r, pl.ds(k * L, L)] = z16
            return c

        lax.fori_loop(0, CHUNK, zstep, 0)
        for t in range(rpt // CHUNK):
            pltpu.sync_copy(rows_v.at[0],
                            acc_sh.at[pl.ds(sid * rpt + t * CHUNK, CHUNK)])

        if with_deg:
            hist = rows_v.at[0, pl.ds(0, nrow)]

            def istep(i, c):
                iota_v[pl.ds(i * L, L)] = lax.iota(jnp.int32, L) + i * L
                return c

            lax.fori_loop(0, nrow // L, istep, 0)

            @pl.when(sid == 0)
            def _():
                pltpu.sync_copy(hist, deg_sh)

        plsc.subcore_barrier()

        if with_deg:
            ones16 = jnp.ones((L,), jnp.float32)
            zeros16i = jnp.zeros((L,), jnp.int32)
            for p in range(NPHASE):
                pltpu.sync_copy(dstc.at[pl.ds(wid * cpw + p * hc, hc)], dst_v)

                def dstep(j, c):
                    for k in range(CHUNK // L):
                        idx = dst_v[j, pl.ds(k * L, L)]
                        plsc.addupdate_scatter(
                            rows_v,
                            [zeros16i, lax.shift_right_logical(idx, 7),
                             idx & 127],
                            ones16)
                    return c

                lax.fori_loop(0, hc, dstep, 0)
            pltpu.sync_copy(hist, deg_sh.at[iota_v], add=True)

        for p in range(NPHASE):
            pltpu.sync_copy(srcc.at[pl.ds(wid * cpw + p * hc, hc)], src_v)
            pltpu.sync_copy(dstc.at[pl.ds(wid * cpw + p * hc, hc)], dst_v)
            for b in range(NBUF):
                pltpu.async_copy(feats.at[src_v.at[b]], rows_v.at[b], gsem[b])

            def step(i, c):
                t = i * NBUF
                for b in range(NBUF):
                    j = t + b
                    pltpu.make_async_copy(
                        feats.at[src_v.at[j]], rows_v.at[b], gsem[b]).wait()
                    pltpu.async_copy(rows_v.at[b], acc_sh.at[dst_v.at[j]],
                                     ssem[b], add=True)
                for b in range(NBUF):
                    j = t + b
                    pltpu.make_async_copy(
                        rows_v.at[b], acc_sh.at[dst_v.at[j]], ssem[b]).wait()
                    jn = j + NBUF

                    @pl.when(jn < hc)
                    def _():
                        pltpu.async_copy(feats.at[src_v.at[jn]],
                                         rows_v.at[b], gsem[b])
                return c

            lax.fori_loop(0, hc // NBUF, step, 0)

        plsc.subcore_barrier()

        pltpu.sync_copy(acc_sh.at[pl.ds(sid * rpt, rpt)],
                        out.at[cid, pl.ds(sid * rpt, rpt)])
        if with_deg:
            @pl.when(sid == 0)
            def _():
                pltpu.sync_copy(deg_sh, out_deg.at[cid])

    return sc_agg


@functools.lru_cache(maxsize=None)
def _make_tc_layer(n, n_pad, rows):
    grid = (n // rows,)

    def body(h_ref, acc_ref, deg_ref, ws_ref, wn_ref, b_ref, out_ref):
        agg = acc_ref[0] + acc_ref[1]
        deg = deg_ref[0] + deg_ref[1]
        mean = agg / jnp.maximum(deg, 1.0)
        o = h_ref[...] @ ws_ref[...] + mean @ wn_ref[...] + b_ref[...]
        out_ref[...] = jnp.maximum(o, 0.0)

    return pl.pallas_call(
        body,
        grid=grid,
        in_specs=[
            pl.BlockSpec((rows, D), lambda i: (i, 0)),
            pl.BlockSpec((2, rows, D), lambda i: (0, i, 0)),
            pl.BlockSpec((2, rows, 1), lambda i: (0, i, 0)),
            pl.BlockSpec((D, D), lambda i: (0, 0)),
            pl.BlockSpec((D, D), lambda i: (0, 0)),
            pl.BlockSpec((1, D), lambda i: (0, 0)),
        ],
        out_specs=pl.BlockSpec((rows, D), lambda i: (i, 0)),
        out_shape=jax.ShapeDtypeStruct((n, D), jnp.float32),
    )


def kernel(x, edge_index, W_self0, W_neigh0, b0, W_self1, W_neigh1, b1):
    n = x.shape[0]
    e = edge_index.shape[1]
    n_pad = _round_up(n + 1, NSUB * L)
    e_pad = _round_up(e, NW * CHUNK * NPHASE * 8)
    cpw = e_pad // (NW * CHUNK)

    src = edge_index[0]
    dst = edge_index[1]
    pad = e_pad - e
    srcc = jnp.concatenate([src, jnp.zeros((pad,), jnp.int32)]).reshape(-1, CHUNK)
    dstc = jnp.concatenate([dst, jnp.full((pad,), n, jnp.int32)]).reshape(-1, CHUNK)

    sc_agg1 = _make_sc_agg(n_pad, cpw, True)
    sc_agg2 = _make_sc_agg(n_pad, cpw, False)
    rows = 2000 if n % 2000 == 0 else n
    tc_layer = _make_tc_layer(n, n_pad, rows)

    agg1, deg = sc_agg1(x, srcc, dstc)
    deg = deg.reshape(2, n_pad, 1)
    h1 = tc_layer(x, agg1, deg, W_self0, W_neigh0, b0.reshape(1, D))
    (agg2,) = sc_agg2(h1, srcc, dstc)
    h2 = tc_layer(h1, agg2, deg, W_self1, W_neigh1, b1.reshape(1, D))
    return h2


# speedup vs baseline: 3.5577x; 1.0289x over previous
import functools

import jax
import jax.numpy as jnp
from jax import lax
from jax.experimental import pallas as pl
from jax.experimental.pallas import tpu as pltpu
from jax.experimental.pallas import tpu_sc as plsc

D = 128
CHUNK = 64
NW = 32
NSUB = 16
L = 16
NBUF = 4
NPHASE = 4


def _round_up(a, b):
    return (a + b - 1) // b * b


@functools.lru_cache(maxsize=None)
def _make_sc_agg(n_pad, cpw, with_deg):
    rpt = n_pad // NSUB
    mesh = plsc.VectorSubcoreMesh(core_axis_name="c", subcore_axis_name="s")

    out_type = [jax.ShapeDtypeStruct((2, n_pad, D), jnp.float32)]
    hc = cpw // NPHASE
    scratch = [
        pltpu.VMEM((hc, CHUNK), jnp.int32),
        pltpu.VMEM((hc, CHUNK), jnp.int32),
        pltpu.VMEM((NBUF, CHUNK, D), jnp.float32),
        pltpu.VMEM_SHARED((n_pad, D), jnp.float32),
    ] + [pltpu.SemaphoreType.DMA] * (2 * NBUF)
    nrow = n_pad // 128
    if with_deg:
        out_type.append(jax.ShapeDtypeStruct((2, nrow, 128), jnp.float32))
        scratch += [
            pltpu.VMEM_SHARED((nrow, 128), jnp.float32),
            pltpu.VMEM((nrow,), jnp.int32),
        ]

    @functools.partial(pl.kernel, out_type=out_type, mesh=mesh,
                       scratch_types=scratch,
                       compiler_params=pltpu.CompilerParams(
                           needs_layout_passes=False))
    def sc_agg(feats, srcc, dstc, out, *rest):
        if with_deg:
            out_deg = rest[0]
            rest = rest[1:]
        src_v, dst_v, rows_v, acc_sh = rest[:4]
        gsem = rest[4:4 + NBUF]
        ssem = rest[4 + NBUF:4 + 2 * NBUF]
        if with_deg:
            deg_sh, iota_v = rest[4 + 2 * NBUF:]
        cid = lax.axis_index("c")
        sid = lax.axis_index("s")
        wid = sid * 2 + cid

        z16 = jnp.zeros((L,), jnp.float32)

        def zstep(r, c):
            for k in range(D // L):
                rows_v[0, r, pl.ds(k * L, L)] = z16
            return c

        lax.fori_loop(0, CHUNK, zstep, 0)
        for t in range(rpt // CHUNK):
            pltpu.sync_copy(rows_v.at[0],
                            acc_sh.at[pl.ds(sid * rpt + t * CHUNK, CHUNK)])

        if with_deg:
            def hzstep(r, c):
                for k in range(D // L):
                    rows_v[1, r, pl.ds(k * L, L)] = z16
                return c

            lax.fori_loop(0, nrow - CHUNK, hzstep, 0)

            def istep(i, c):
                iota_v[pl.ds(i * L, L)] = lax.iota(jnp.int32, L) + i * L
                return c

            lax.fori_loop(0, nrow // L, istep, 0)

            @pl.when(sid == 0)
            def _():
                pltpu.sync_copy(rows_v.at[0], deg_sh.at[pl.ds(0, CHUNK)])
                pltpu.sync_copy(rows_v.at[1, pl.ds(0, nrow - CHUNK)],
                                deg_sh.at[pl.ds(CHUNK, nrow - CHUNK)])

        plsc.subcore_barrier()

        if with_deg:
            ones16 = jnp.ones((L,), jnp.float32)
            for p in range(NPHASE):
                pltpu.sync_copy(dstc.at[pl.ds(wid * cpw + p * hc, hc)], dst_v)

                def dstep(j, c):
                    for k in range(CHUNK // L):
                        idx = dst_v[j, pl.ds(k * L, L)]
                        fr = lax.shift_right_logical(idx, 7)
                        plsc.addupdate_scatter(
                            rows_v,
                            [fr // CHUNK, fr % CHUNK, idx & 127],
                            ones16)
                    return c

                lax.fori_loop(0, hc, dstep, 0)
            pltpu.sync_copy(rows_v.at[0],
                            deg_sh.at[iota_v.at[pl.ds(0, CHUNK)]], add=True)
            pltpu.sync_copy(rows_v.at[1, pl.ds(0, nrow - CHUNK)],
                            deg_sh.at[iota_v.at[pl.ds(CHUNK, nrow - CHUNK)]],
                            add=True)

        for p in range(NPHASE):
            pltpu.sync_copy(srcc.at[pl.ds(wid * cpw + p * hc, hc)], src_v)
            pltpu.sync_copy(dstc.at[pl.ds(wid * cpw + p * hc, hc)], dst_v)
            for b in range(NBUF):
                pltpu.async_copy(feats.at[src_v.at[b]], rows_v.at[b], gsem[b])

            def step(i, c):
                t = i * NBUF
                for b in range(NBUF):
                    j = t + b
                    pltpu.make_async_copy(
                        feats.at[src_v.at[j]], rows_v.at[b], gsem[b]).wait()
                    pltpu.async_copy(rows_v.at[b], acc_sh.at[dst_v.at[j]],
                                     ssem[b], add=True)
                for b in range(NBUF):
                    j = t + b
                    pltpu.make_async_copy(
                        rows_v.at[b], acc_sh.at[dst_v.at[j]], ssem[b]).wait()
                    jn = j + NBUF

                    @pl.when(jn < hc)
                    def _():
                        pltpu.async_copy(feats.at[src_v.at[jn]],
                                         rows_v.at[b], gsem[b])
                return c

            lax.fori_loop(0, hc // NBUF, step, 0)

        plsc.subcore_barrier()

        pltpu.sync_copy(acc_sh.at[pl.ds(sid * rpt, rpt)],
                        out.at[cid, pl.ds(sid * rpt, rpt)])
        if with_deg:
            @pl.when(sid == 0)
            def _():
                pltpu.sync_copy(deg_sh, out_deg.at[cid])

    return sc_agg


@functools.lru_cache(maxsize=None)
def _make_tc_layer(n, n_pad, rows):
    grid = (n // rows,)

    def body(h_ref, acc_ref, deg_ref, ws_ref, wn_ref, b_ref, out_ref):
        agg = acc_ref[0] + acc_ref[1]
        deg = deg_ref[0] + deg_ref[1]
        mean = agg / jnp.maximum(deg, 1.0)
        o = h_ref[...] @ ws_ref[...] + mean @ wn_ref[...] + b_ref[...]
        out_ref[...] = jnp.maximum(o, 0.0)

    return pl.pallas_call(
        body,
        grid=grid,
        in_specs=[
            pl.BlockSpec((rows, D), lambda i: (i, 0)),
            pl.BlockSpec((2, rows, D), lambda i: (0, i, 0)),
            pl.BlockSpec((2, rows, 1), lambda i: (0, i, 0)),
            pl.BlockSpec((D, D), lambda i: (0, 0)),
            pl.BlockSpec((D, D), lambda i: (0, 0)),
            pl.BlockSpec((1, D), lambda i: (0, 0)),
        ],
        out_specs=pl.BlockSpec((rows, D), lambda i: (i, 0)),
        out_shape=jax.ShapeDtypeStruct((n, D), jnp.float32),
    )


def kernel(x, edge_index, W_self0, W_neigh0, b0, W_self1, W_neigh1, b1):
    n = x.shape[0]
    e = edge_index.shape[1]
    n_pad = _round_up(n + 1, NSUB * L)
    e_pad = _round_up(e, NW * CHUNK * NPHASE * 8)
    cpw = e_pad // (NW * CHUNK)

    src = edge_index[0]
    dst = edge_index[1]
    pad = e_pad - e
    srcc = jnp.concatenate([src, jnp.zeros((pad,), jnp.int32)]).reshape(-1, CHUNK)
    dstc = jnp.concatenate([dst, jnp.full((pad,), n, jnp.int32)]).reshape(-1, CHUNK)

    sc_agg1 = _make_sc_agg(n_pad, cpw, True)
    sc_agg2 = _make_sc_agg(n_pad, cpw, False)
    rows = 2000 if n % 2000 == 0 else n
    tc_layer = _make_tc_layer(n, n_pad, rows)

    agg1, deg = sc_agg1(x, srcc, dstc)
    deg = deg.reshape(2, n_pad, 1)
    h1 = tc_layer(x, agg1, deg, W_self0, W_neigh0, b0.reshape(1, D))
    (agg2,) = sc_agg2(h1, srcc, dstc)
    h2 = tc_layer(h1, agg2, deg, W_self1, W_neigh1, b1.reshape(1, D))
    return h2


# speedup vs baseline: 3.5837x; 1.0073x over previous
import functools

import jax
import jax.numpy as jnp
from jax import lax
from jax.experimental import pallas as pl
from jax.experimental.pallas import tpu as pltpu
from jax.experimental.pallas import tpu_sc as plsc

D = 128
CHUNK = 64
NW = 32
NSUB = 16
L = 16
NBUF = 4
NPHASE = 4


def _round_up(a, b):
    return (a + b - 1) // b * b


@functools.lru_cache(maxsize=None)
def _make_sc_agg(n_pad, cpw, with_deg, nbuf):
    rpt = n_pad // NSUB
    mesh = plsc.VectorSubcoreMesh(core_axis_name="c", subcore_axis_name="s")

    out_type = [jax.ShapeDtypeStruct((2, n_pad, D), jnp.float32)]
    hc = cpw // NPHASE
    scratch = [
        pltpu.VMEM((hc, CHUNK), jnp.int32),
        pltpu.VMEM((hc, CHUNK), jnp.int32),
        pltpu.VMEM((nbuf, CHUNK, D), jnp.float32),
        pltpu.VMEM_SHARED((n_pad, D), jnp.float32),
    ] + [pltpu.SemaphoreType.DMA] * (2 * nbuf)
    nrow = n_pad // 128
    if with_deg:
        out_type.append(jax.ShapeDtypeStruct((2, nrow, 128), jnp.float32))
        scratch += [
            pltpu.VMEM_SHARED((nrow, 128), jnp.float32),
            pltpu.VMEM((nrow, 128), jnp.float32),
            pltpu.VMEM((nrow,), jnp.int32),
        ]

    @functools.partial(pl.kernel, out_type=out_type, mesh=mesh,
                       scratch_types=scratch,
                       compiler_params=pltpu.CompilerParams(
                           needs_layout_passes=False))
    def sc_agg(feats, srcc, dstc, out, *rest):
        if with_deg:
            out_deg = rest[0]
            rest = rest[1:]
        src_v, dst_v, rows_v, acc_sh = rest[:4]
        gsem = rest[4:4 + nbuf]
        ssem = rest[4 + nbuf:4 + 2 * nbuf]
        if with_deg:
            deg_sh, hist_v, iota_v = rest[4 + 2 * nbuf:]
        cid = lax.axis_index("c")
        sid = lax.axis_index("s")
        wid = sid * 2 + cid

        z16 = jnp.zeros((L,), jnp.float32)

        def zstep(r, c):
            for k in range(D // L):
                rows_v[0, r, pl.ds(k * L, L)] = z16
            return c

        lax.fori_loop(0, CHUNK, zstep, 0)
        for t in range(rpt // CHUNK):
            pltpu.sync_copy(rows_v.at[0],
                            acc_sh.at[pl.ds(sid * rpt + t * CHUNK, CHUNK)])

        if with_deg:
            ones16 = jnp.ones((L,), jnp.float32)

            def hzstep(r, c):
                for k in range(D // L):
                    hist_v[r, pl.ds(k * L, L)] = z16
                return c

            lax.fori_loop(0, nrow, hzstep, 0)

            def istep(i, c):
                iota_v[pl.ds(i * L, L)] = lax.iota(jnp.int32, L) + i * L
                return c

            lax.fori_loop(0, nrow // L, istep, 0)

            @pl.when(sid == 0)
            def _():
                pltpu.sync_copy(hist_v, deg_sh)

        plsc.subcore_barrier()

        base = hc // nbuf * nbuf
        for p in range(NPHASE):
            pltpu.sync_copy(srcc.at[pl.ds(wid * cpw + p * hc, hc)], src_v)
            pltpu.sync_copy(dstc.at[pl.ds(wid * cpw + p * hc, hc)], dst_v)
            for b in range(nbuf):
                pltpu.async_copy(feats.at[src_v.at[b]], rows_v.at[b], gsem[b])

            def step(i, c):
                t = i * nbuf
                for b in range(nbuf):
                    j = t + b
                    pltpu.make_async_copy(
                        feats.at[src_v.at[j]], rows_v.at[b], gsem[b]).wait()
                    pltpu.async_copy(rows_v.at[b], acc_sh.at[dst_v.at[j]],
                                     ssem[b], add=True)
                    if with_deg:
                        for k in range(CHUNK // L):
                            idx = dst_v[j, pl.ds(k * L, L)]
                            fr = lax.shift_right_logical(idx, 7)
                            plsc.addupdate_scatter(
                                hist_v, [fr, idx & 127], ones16)
                for b in range(nbuf):
                    j = t + b
                    pltpu.make_async_copy(
                        rows_v.at[b], acc_sh.at[dst_v.at[j]], ssem[b]).wait()
                    jn = j + nbuf

                    @pl.when(jn < hc)
                    def _():
                        pltpu.async_copy(feats.at[src_v.at[jn]],
                                         rows_v.at[b], gsem[b])
                return c

            lax.fori_loop(0, hc // nbuf, step, 0)

            for r in range(hc - base):
                j = base + r
                pltpu.make_async_copy(
                    feats.at[src_v.at[j]], rows_v.at[r], gsem[r]).wait()
                pltpu.async_copy(rows_v.at[r], acc_sh.at[dst_v.at[j]],
                                 ssem[r], add=True)
                if with_deg:
                    for k in range(CHUNK // L):
                        idx = dst_v[j, pl.ds(k * L, L)]
                        fr = lax.shift_right_logical(idx, 7)
                        plsc.addupdate_scatter(
                            hist_v, [fr, idx & 127], ones16)
            for r in range(hc - base):
                j = base + r
                pltpu.make_async_copy(
                    rows_v.at[r], acc_sh.at[dst_v.at[j]], ssem[r]).wait()

        if with_deg:
            pltpu.sync_copy(hist_v,
                            deg_sh.at[iota_v.at[pl.ds(0, nrow)]], add=True)

        plsc.subcore_barrier()

        pltpu.sync_copy(acc_sh.at[pl.ds(sid * rpt, rpt)],
                        out.at[cid, pl.ds(sid * rpt, rpt)])
        if with_deg:
            @pl.when(sid == 0)
            def _():
                pltpu.sync_copy(deg_sh, out_deg.at[cid])

    return sc_agg


@functools.lru_cache(maxsize=None)
def _make_tc_layer(n, n_pad, rows):
    grid = (n // rows,)

    def body(h_ref, acc_ref, deg_ref, ws_ref, wn_ref, b_ref, out_ref):
        agg = acc_ref[0] + acc_ref[1]
        deg = deg_ref[0] + deg_ref[1]
        mean = agg / jnp.maximum(deg, 1.0)
        o = h_ref[...] @ ws_ref[...] + mean @ wn_ref[...] + b_ref[...]
        out_ref[...] = jnp.maximum(o, 0.0)

    return pl.pallas_call(
        body,
        grid=grid,
        in_specs=[
            pl.BlockSpec((rows, D), lambda i: (i, 0)),
            pl.BlockSpec((2, rows, D), lambda i: (0, i, 0)),
            pl.BlockSpec((2, rows, 1), lambda i: (0, i, 0)),
            pl.BlockSpec((D, D), lambda i: (0, 0)),
            pl.BlockSpec((D, D), lambda i: (0, 0)),
            pl.BlockSpec((1, D), lambda i: (0, 0)),
        ],
        out_specs=pl.BlockSpec((rows, D), lambda i: (i, 0)),
        out_shape=jax.ShapeDtypeStruct((n, D), jnp.float32),
    )


def kernel(x, edge_index, W_self0, W_neigh0, b0, W_self1, W_neigh1, b1):
    n = x.shape[0]
    e = edge_index.shape[1]
    n_pad = _round_up(n + 1, NSUB * L)
    e_pad = _round_up(e, NW * CHUNK * NPHASE * 8)
    cpw = e_pad // (NW * CHUNK)

    src = edge_index[0]
    dst = edge_index[1]
    pad = e_pad - e
    srcc = jnp.concatenate([src, jnp.zeros((pad,), jnp.int32)]).reshape(-1, CHUNK)
    dstc = jnp.concatenate([dst, jnp.full((pad,), n, jnp.int32)]).reshape(-1, CHUNK)

    sc_agg1 = _make_sc_agg(n_pad, cpw, True, 3)
    sc_agg2 = _make_sc_agg(n_pad, cpw, False, NBUF)
    rows = 2000 if n % 2000 == 0 else n
    tc_layer = _make_tc_layer(n, n_pad, rows)

    agg1, deg = sc_agg1(x, srcc, dstc)
    deg = deg.reshape(2, n_pad, 1)
    h1 = tc_layer(x, agg1, deg, W_self0, W_neigh0, b0.reshape(1, D))
    (agg2,) = sc_agg2(h1, srcc, dstc)
    h2 = tc_layer(h1, agg2, deg, W_self1, W_neigh1, b1.reshape(1, D))
    return h2


# speedup vs baseline: 3.5854x; 1.0005x over previous
import functools

import jax
import jax.numpy as jnp
from jax import lax
from jax.experimental import pallas as pl
from jax.experimental.pallas import tpu as pltpu
from jax.experimental.pallas import tpu_sc as plsc

D = 128
CHUNK = 64
NW = 32
NSUB = 16
L = 16
NBUF = 4
NPHASE = 4


def _round_up(a, b):
    return (a + b - 1) // b * b


@functools.lru_cache(maxsize=None)
def _make_sc_agg(n_pad, cpw, with_deg, nbuf, nphase):
    rpt = n_pad // NSUB
    mesh = plsc.VectorSubcoreMesh(core_axis_name="c", subcore_axis_name="s")

    out_type = [jax.ShapeDtypeStruct((2, n_pad, D), jnp.float32)]
    hc = cpw // nphase
    scratch = [
        pltpu.VMEM((hc, CHUNK), jnp.int32),
        pltpu.VMEM((hc, CHUNK), jnp.int32),
        pltpu.VMEM((nbuf, CHUNK, D), jnp.float32),
        pltpu.VMEM_SHARED((n_pad, D), jnp.float32),
    ] + [pltpu.SemaphoreType.DMA] * (2 * nbuf)
    nrow = n_pad // 128
    if with_deg:
        out_type.append(jax.ShapeDtypeStruct((2, nrow, 128), jnp.float32))
        scratch += [
            pltpu.VMEM_SHARED((nrow, 128), jnp.float32),
            pltpu.VMEM((nrow, 128), jnp.float32),
            pltpu.VMEM((nrow,), jnp.int32),
        ]

    @functools.partial(pl.kernel, out_type=out_type, mesh=mesh,
                       scratch_types=scratch,
                       compiler_params=pltpu.CompilerParams(
                           needs_layout_passes=False))
    def sc_agg(feats, srcc, dstc, out, *rest):
        if with_deg:
            out_deg = rest[0]
            rest = rest[1:]
        src_v, dst_v, rows_v, acc_sh = rest[:4]
        gsem = rest[4:4 + nbuf]
        ssem = rest[4 + nbuf:4 + 2 * nbuf]
        if with_deg:
            deg_sh, hist_v, iota_v = rest[4 + 2 * nbuf:]
        cid = lax.axis_index("c")
        sid = lax.axis_index("s")
        wid = sid * 2 + cid

        z16 = jnp.zeros((L,), jnp.float32)

        def zstep(r, c):
            for k in range(D // L):
                rows_v[0, r, pl.ds(k * L, L)] = z16
            return c

        lax.fori_loop(0, CHUNK, zstep, 0)
        for t in range(rpt // CHUNK):
            pltpu.sync_copy(rows_v.at[0],
                            acc_sh.at[pl.ds(sid * rpt + t * CHUNK, CHUNK)])

        if with_deg:
            ones16 = jnp.ones((L,), jnp.float32)

            def hzstep(r, c):
                for k in range(D // L):
                    hist_v[r, pl.ds(k * L, L)] = z16
                return c

            lax.fori_loop(0, nrow, hzstep, 0)

            def istep(i, c):
                iota_v[pl.ds(i * L, L)] = lax.iota(jnp.int32, L) + i * L
                return c

            lax.fori_loop(0, nrow // L, istep, 0)

            @pl.when(sid == 0)
            def _():
                pltpu.sync_copy(hist_v, deg_sh)

        plsc.subcore_barrier()

        base = hc // nbuf * nbuf
        for p in range(nphase):
            pltpu.sync_copy(srcc.at[pl.ds(wid * cpw + p * hc, hc)], src_v)
            pltpu.sync_copy(dstc.at[pl.ds(wid * cpw + p * hc, hc)], dst_v)
            for b in range(nbuf):
                pltpu.async_copy(feats.at[src_v.at[b]], rows_v.at[b], gsem[b])

            def step(i, c):
                t = i * nbuf
                for b in range(nbuf):
                    j = t + b
                    pltpu.make_async_copy(
                        feats.at[src_v.at[j]], rows_v.at[b], gsem[b]).wait()
                    pltpu.async_copy(rows_v.at[b], acc_sh.at[dst_v.at[j]],
                                     ssem[b], add=True)
                    if with_deg:
                        for k in range(CHUNK // L):
                            idx = dst_v[j, pl.ds(k * L, L)]
                            fr = lax.shift_right_logical(idx, 7)
                            plsc.addupdate_scatter(
                                hist_v, [fr, idx & 127], ones16)
                for b in range(nbuf):
                    j = t + b
                    pltpu.make_async_copy(
                        rows_v.at[b], acc_sh.at[dst_v.at[j]], ssem[b]).wait()
                    jn = j + nbuf

                    @pl.when(jn < hc)
                    def _():
                        pltpu.async_copy(feats.at[src_v.at[jn]],
                                         rows_v.at[b], gsem[b])
                return c

            lax.fori_loop(0, hc // nbuf, step, 0)

            for r in range(hc - base):
                j = base + r
                pltpu.make_async_copy(
                    feats.at[src_v.at[j]], rows_v.at[r], gsem[r]).wait()
                pltpu.async_copy(rows_v.at[r], acc_sh.at[dst_v.at[j]],
                                 ssem[r], add=True)
                if with_deg:
                    for k in range(CHUNK // L):
                        idx = dst_v[j, pl.ds(k * L, L)]
                        fr = lax.shift_right_logical(idx, 7)
                        plsc.addupdate_scatter(
                            hist_v, [fr, idx & 127], ones16)
            for r in range(hc - base):
                j = base + r
                pltpu.make_async_copy(
                    rows_v.at[r], acc_sh.at[dst_v.at[j]], ssem[r]).wait()

        if with_deg:
            pltpu.sync_copy(hist_v,
                            deg_sh.at[iota_v.at[pl.ds(0, nrow)]], add=True)

        plsc.subcore_barrier()

        pltpu.sync_copy(acc_sh.at[pl.ds(sid * rpt, rpt)],
                        out.at[cid, pl.ds(sid * rpt, rpt)])
        if with_deg:
            @pl.when(sid == 0)
            def _():
                pltpu.sync_copy(deg_sh, out_deg.at[cid])

    return sc_agg


@functools.lru_cache(maxsize=None)
def _make_tc_layer(n, n_pad, rows):
    grid = (n // rows,)

    def body(h_ref, acc_ref, deg_ref, ws_ref, wn_ref, b_ref, out_ref):
        agg = acc_ref[0] + acc_ref[1]
        deg = deg_ref[0] + deg_ref[1]
        mean = agg / jnp.maximum(deg, 1.0)
        o = h_ref[...] @ ws_ref[...] + mean @ wn_ref[...] + b_ref[...]
        out_ref[...] = jnp.maximum(o, 0.0)

    return pl.pallas_call(
        body,
        grid=grid,
        in_specs=[
            pl.BlockSpec((rows, D), lambda i: (i, 0)),
            pl.BlockSpec((2, rows, D), lambda i: (0, i, 0)),
            pl.BlockSpec((2, rows, 1), lambda i: (0, i, 0)),
            pl.BlockSpec((D, D), lambda i: (0, 0)),
            pl.BlockSpec((D, D), lambda i: (0, 0)),
            pl.BlockSpec((1, D), lambda i: (0, 0)),
        ],
        out_specs=pl.BlockSpec((rows, D), lambda i: (i, 0)),
        out_shape=jax.ShapeDtypeStruct((n, D), jnp.float32),
    )


def kernel(x, edge_index, W_self0, W_neigh0, b0, W_self1, W_neigh1, b1):
    n = x.shape[0]
    e = edge_index.shape[1]
    n_pad = _round_up(n + 1, NSUB * L)
    e_pad = _round_up(e, NW * CHUNK * NPHASE * 8)
    cpw = e_pad // (NW * CHUNK)

    src = edge_index[0]
    dst = edge_index[1]
    pad = e_pad - e
    srcc = jnp.concatenate([src, jnp.zeros((pad,), jnp.int32)]).reshape(-1, CHUNK)
    dstc = jnp.concatenate([dst, jnp.full((pad,), n, jnp.int32)]).reshape(-1, CHUNK)

    sc_agg1 = _make_sc_agg(n_pad, cpw, True, 3, NPHASE)
    sc_agg2 = _make_sc_agg(n_pad, cpw, False, NBUF, NPHASE)
    rows = 2000 if n % 2000 == 0 else n
    tc_layer = _make_tc_layer(n, n_pad, rows)

    agg1, deg = sc_agg1(x, srcc, dstc)
    deg = deg.reshape(2, n_pad, 1)
    h1 = tc_layer(x, agg1, deg, W_self0, W_neigh0, b0.reshape(1, D))
    (agg2,) = sc_agg2(h1, srcc, dstc)
    h2 = tc_layer(h1, agg2, deg, W_self1, W_neigh1, b1.reshape(1, D))
    return h2
